# baseline (speedup 1.0000x reference)
import functools

import jax
import jax.numpy as jnp
from jax import lax
from jax.experimental import pallas as pl
from jax.experimental.pallas import tpu as pltpu
from jax.experimental.pallas import tpu_sc as plsc

B = 16384
S_ES = 64
DPAIR = 128
NC = 2
NS = 16
L = 16
NW = NC * NS
BPW = B // NW
K = 16
NCHUNK = BPW // K

_S0 = 9.99996152e-01
_S1 = -1.66647032e-01
_S2 = 8.31724544e-03
_S3 = -1.93765902e-04
_S4 = 2.19812516e-06
_TWO_PI = 6.283185307179586
_INV_2PI = 0.15915494309189535
_RND = 12582912.0


def _sin(x):
    k = (x * _INV_2PI + _RND) - _RND
    xr = x - k * _TWO_PI
    s = xr * xr
    p = _S4
    p = p * s + _S3
    p = p * s + _S2
    p = p * s + _S1
    p = p * s + _S0
    return p * xr


_EB = 1024
_NEB = (100000 + _EB - 1) // _EB


def _make_tpose(npair):
    def body(*refs):
        ins = refs[:2 * npair]
        outs = refs[2 * npair:]
        for p in range(npair):
            outs[p][:, 0:S_ES] = ins[2 * p][...].T
            outs[p][:, S_ES:DPAIR] = ins[2 * p + 1][...].T

    in_specs = [pl.BlockSpec((S_ES, _EB), lambda i: (0, i))
                for _ in range(2 * npair)]
    out_specs = [pl.BlockSpec((_EB, DPAIR), lambda i: (i, 0))
                 for _ in range(npair)]
    out_shape = [jax.ShapeDtypeStruct((100000, DPAIR), jnp.float32)
                 for _ in range(npair)]
    return pl.pallas_call(body, grid=(_NEB,), in_specs=in_specs,
                          out_specs=out_specs, out_shape=out_shape)


def _worker_prelude(s_h, o_h, si_v, oi_v):
    wid = lax.axis_index("s") * NC + lax.axis_index("c")
    base = wid * BPW
    pltpu.sync_copy(s_h.at[pl.ds(base, BPW)], si_v)
    pltpu.sync_copy(o_h.at[pl.ds(base, BPW)], oi_v)
    return base


def _pipeline(fire, drain, compute):
    fire(0, 0)

    def loop_body(g, carry):
        c0 = 2 * g
        c1 = 2 * g + 1
        fire(c1 * K, 1)
        drain(c0 * K, 0)
        compute(c0 * K, 0)
        nxt0 = jnp.minimum(c0 + 2, NCHUNK - 1) * K
        fire(nxt0, 0)
        drain(c1 * K, 1)
        compute(c1 * K, 1)
        return carry

    lax.fori_loop(0, NCHUNK // 2, loop_body, 0)
    drain((NCHUNK - 1) * K, 0)


def _kT_body(s_h, o_h, d_h, h_h, frq_h, phi_h, amp_h, out_h, *sc):
    si_v, oi_v, d_v, h_v = sc[0:4]
    sets = [sc[4 + t * 7:4 + (t + 1) * 7] for t in range(2)]
    sems = sc[18:20]

    base = _worker_prelude(s_h, o_h, si_v, oi_v)
    pltpu.sync_copy(d_h.at[pl.ds(base, BPW)], d_v)
    pltpu.sync_copy(h_h.at[pl.ds(base, BPW)], h_v)

    def _descs(cbase, t):
        bufs, sem = sets[t], sems[t]
        si = si_v.at[pl.ds(cbase, K)]
        oi = oi_v.at[pl.ds(cbase, K)]
        return [(frq_h.at[si], bufs[0], sem), (phi_h.at[si], bufs[1], sem),
                (amp_h.at[si], bufs[2], sem), (frq_h.at[oi], bufs[3], sem),
                (phi_h.at[oi], bufs[4], sem), (amp_h.at[oi], bufs[5], sem)]

    def fire(cbase, t):
        for src, dst, sem in _descs(cbase, t):
            pltpu.async_copy(src, dst, sem)

    def drain(cbase, t):
        for src, dst, sem in _descs(cbase, t):
            pltpu.make_async_copy(src, dst, sem).wait()

    def compute(cbase, t):
        bufs = sets[t]
        ob = bufs[6]

        def ex_body(l, carry):
            idxv = lax.broadcast(cbase + l, (L,))
            db = plsc.load_gather(d_v, [idxv])
            hb = plsc.load_gather(h_v, [idxv])
            for j in range(S_ES // L):
                lo = pl.ds(j * L, L)
                hi = pl.ds(S_ES + j * L, L)
                ts = (bufs[2][l, lo] * _sin(db * bufs[0][l, lo] + bufs[1][l, lo])
                      + bufs[2][l, hi] * _sin(hb * bufs[0][l, hi] + bufs[1][l, hi]))
                to = (bufs[5][l, lo] * _sin(db * bufs[3][l, lo] + bufs[4][l, lo])
                      + bufs[5][l, hi] * _sin(hb * bufs[3][l, hi] + bufs[4][l, hi]))
                ob[l, lo] = ts
                ob[l, hi] = to
            return carry

        lax.fori_loop(0, L, ex_body, 0)
        pltpu.sync_copy(ob, out_h.at[pl.ds(base + cbase, K)])

    _pipeline(fire, drain, compute)


def _make_kT(mesh, params):
    scratch = (
        [pltpu.VMEM((BPW,), jnp.int32)] * 2
        + [pltpu.VMEM((BPW,), jnp.float32)] * 2
        + [pltpu.VMEM((K, DPAIR), jnp.float32)] * 14
        + [pltpu.SemaphoreType.DMA] * 2
    )
    return pl.kernel(
        _kT_body,
        out_type=jax.ShapeDtypeStruct((B, DPAIR), jnp.float32),
        mesh=mesh, scratch_types=scratch, compiler_params=params)


def _kF_body(s_h, o_h, r_h, ep_h, rf_h, ri_h, ts_h, to_h, out_h, *sc):
    si_v, oi_v, ri_v, out_v = sc[0:4]
    sets = [sc[4 + t * 6:4 + (t + 1) * 6] for t in range(2)]
    sems = sc[16:18]

    base = _worker_prelude(s_h, o_h, si_v, oi_v)
    pltpu.sync_copy(r_h.at[pl.ds(base, BPW)], ri_v)

    lane = lax.iota(jnp.int32, L)

    def _descs(cbase, t):
        bufs, sem = sets[t], sems[t]
        si = si_v.at[pl.ds(cbase, K)]
        oi = oi_v.at[pl.ds(cbase, K)]
        ri = ri_v.at[pl.ds(cbase, K)]
        rows = pl.ds(base + cbase, K)
        return [(ep_h.at[si], bufs[0], sem), (ep_h.at[oi], bufs[1], sem),
                (rf_h.at[ri], bufs[2], sem), (ri_h.at[ri], bufs[3], sem),
                (ts_h.at[rows], bufs[4], sem), (to_h.at[rows], bufs[5], sem)]

    def fire(cbase, t):
        for src, dst, sem in _descs(cbase, t):
            pltpu.async_copy(src, dst, sem)

    def drain(cbase, t):
        for src, dst, sem in _descs(cbase, t):
            pltpu.make_async_copy(src, dst, sem).wait()

    def compute(cbase, t):
        epS, epO, bRf, bRi, bTs, bTo = sets[t]

        def ex_body(l, svec):
            acc = jnp.zeros((L,), jnp.float32)
            for j in range(S_ES // L):
                lo = pl.ds(j * L, L)
                hi = pl.ds(S_ES + j * L, L)
                acc = acc + epS[l, lo] * bRf[l, lo] * epO[l, hi]
                acc = acc + bTs[l, lo] * bRf[l, hi] * bTo[l, hi]
                acc = acc + epO[l, lo] * bRi[l, lo] * epS[l, hi]
                acc = acc + bTs[l, hi] * bRi[l, hi] * bTo[l, lo]
            score = 0.5 * jnp.sum(acc)
            return jnp.where(lane == l, score, svec)

        svec = lax.fori_loop(0, L, ex_body, jnp.zeros((L,), jnp.float32))
        out_v[pl.ds(cbase, L)] = svec

    _pipeline(fire, drain, compute)
    pltpu.sync_copy(out_v, out_h.at[pl.ds(base, BPW)])


def _make_kF(mesh, params):
    scratch = (
        [pltpu.VMEM((BPW,), jnp.int32)] * 3
        + [pltpu.VMEM((BPW,), jnp.float32)]
        + [pltpu.VMEM((K, DPAIR), jnp.float32)] * 12
        + [pltpu.SemaphoreType.DMA] * 2
    )
    return pl.kernel(
        _kF_body,
        out_type=jax.ShapeDtypeStruct((B,), jnp.float32),
        mesh=mesh, scratch_types=scratch, compiler_params=params)


@jax.jit
def kernel(s, o, r, t, E_s, E_o, R_f, R_i,
           d_frq_s, d_frq_o, h_frq_s, h_frq_o,
           d_phi_s, d_phi_o, h_phi_s, h_phi_o,
           d_amp_s, d_amp_o, h_amp_s, h_amp_o):
    d = t[:, 0].astype(jnp.float32)
    h = t[:, 1].astype(jnp.float32)

    tpS = _make_tpose(3)(d_frq_s.T, h_frq_s.T, d_phi_s.T, h_phi_s.T,
                         d_amp_s.T, h_amp_s.T)
    tpO = _make_tpose(3)(d_frq_o.T, h_frq_o.T, d_phi_o.T, h_phi_o.T,
                         d_amp_o.T, h_amp_o.T)
    (epair,) = _make_tpose(1)(E_s.T, E_o.T)

    mesh = plsc.VectorSubcoreMesh(
        core_axis_name="c", subcore_axis_name="s", num_cores=NC, num_subcores=NS)
    params = pltpu.CompilerParams(
        needs_layout_passes=False, use_tc_tiling_on_sc=True)

    ts_rows = _make_kT(mesh, params)(s, o, d, h, *tpS)
    to_rows = _make_kT(mesh, params)(s, o, d, h, *tpO)
    return _make_kF(mesh, params)(s, o, r, epair, R_f, R_i, ts_rows, to_rows)


# speedup vs baseline: 1.0003x; 1.0003x over previous
import functools

import jax
import jax.numpy as jnp
from jax import lax
from jax.experimental import pallas as pl
from jax.experimental.pallas import tpu as pltpu
from jax.experimental.pallas import tpu_sc as plsc

B = 16384
S_ES = 64
DPAIR = 128
NC = 2
NS = 16
L = 16
NW = NC * NS
BPW = B // NW
K = 16
NCHUNK = BPW // K

_S0 = 9.99996152e-01
_S1 = -1.66647032e-01
_S2 = 8.31724544e-03
_S3 = -1.93765902e-04
_S4 = 2.19812516e-06
_TWO_PI = 6.283185307179586
_INV_2PI = 0.15915494309189535
_RND = 12582912.0


def _sin(x):
    k = (x * _INV_2PI + _RND) - _RND
    xr = x - k * _TWO_PI
    s = xr * xr
    p = _S4
    p = p * s + _S3
    p = p * s + _S2
    p = p * s + _S1
    p = p * s + _S0
    return p * xr


_EB = 1024
_NEB = (100000 + _EB - 1) // _EB


def _make_tpose(npair):
    def body(*refs):
        ins = refs[:2 * npair]
        outs = refs[2 * npair:]
        for p in range(npair):
            outs[p][:, 0:S_ES] = ins[2 * p][...].T
            outs[p][:, S_ES:DPAIR] = ins[2 * p + 1][...].T

    in_specs = [pl.BlockSpec((S_ES, _EB), lambda i: (0, i))
                for _ in range(2 * npair)]
    out_specs = [pl.BlockSpec((_EB, DPAIR), lambda i: (i, 0))
                 for _ in range(npair)]
    out_shape = [jax.ShapeDtypeStruct((100000, DPAIR), jnp.float32)
                 for _ in range(npair)]
    return pl.pallas_call(
        body, grid=(_NEB,), in_specs=in_specs, out_specs=out_specs,
        out_shape=out_shape,
        compiler_params=pltpu.CompilerParams(dimension_semantics=("parallel",)))


def _worker_prelude(s_h, o_h, si_v, oi_v):
    wid = lax.axis_index("s") * NC + lax.axis_index("c")
    base = wid * BPW
    pltpu.sync_copy(s_h.at[pl.ds(base, BPW)], si_v)
    pltpu.sync_copy(o_h.at[pl.ds(base, BPW)], oi_v)
    return base


def _pipeline(fire, drain, compute):
    fire(0, 0)

    def loop_body(g, carry):
        c0 = 2 * g
        c1 = 2 * g + 1
        fire(c1 * K, 1)
        drain(c0 * K, 0)
        compute(c0 * K, 0)
        nxt0 = jnp.minimum(c0 + 2, NCHUNK - 1) * K
        fire(nxt0, 0)
        drain(c1 * K, 1)
        compute(c1 * K, 1)
        return carry

    lax.fori_loop(0, NCHUNK // 2, loop_body, 0)
    drain((NCHUNK - 1) * K, 0)


def _kT_body(s_h, o_h, d_h, h_h, frq_h, phi_h, amp_h, out_h, *sc):
    si_v, oi_v, d_v, h_v = sc[0:4]
    sets = [sc[4 + t * 7:4 + (t + 1) * 7] for t in range(2)]
    sems = sc[18:20]

    base = _worker_prelude(s_h, o_h, si_v, oi_v)
    pltpu.sync_copy(d_h.at[pl.ds(base, BPW)], d_v)
    pltpu.sync_copy(h_h.at[pl.ds(base, BPW)], h_v)

    def _descs(cbase, t):
        bufs, sem = sets[t], sems[t]
        si = si_v.at[pl.ds(cbase, K)]
        oi = oi_v.at[pl.ds(cbase, K)]
        return [(frq_h.at[si], bufs[0], sem), (phi_h.at[si], bufs[1], sem),
                (amp_h.at[si], bufs[2], sem), (frq_h.at[oi], bufs[3], sem),
                (phi_h.at[oi], bufs[4], sem), (amp_h.at[oi], bufs[5], sem)]

    def fire(cbase, t):
        for src, dst, sem in _descs(cbase, t):
            pltpu.async_copy(src, dst, sem)

    def drain(cbase, t):
        for src, dst, sem in _descs(cbase, t):
            pltpu.make_async_copy(src, dst, sem).wait()

    def compute(cbase, t):
        bufs = sets[t]
        ob = bufs[6]

        def ex_body(l, carry):
            idxv = lax.broadcast(cbase + l, (L,))
            db = plsc.load_gather(d_v, [idxv])
            hb = plsc.load_gather(h_v, [idxv])
            for j in range(S_ES // L):
                lo = pl.ds(j * L, L)
                hi = pl.ds(S_ES + j * L, L)
                ts = (bufs[2][l, lo] * _sin(db * bufs[0][l, lo] + bufs[1][l, lo])
                      + bufs[2][l, hi] * _sin(hb * bufs[0][l, hi] + bufs[1][l, hi]))
                to = (bufs[5][l, lo] * _sin(db * bufs[3][l, lo] + bufs[4][l, lo])
                      + bufs[5][l, hi] * _sin(hb * bufs[3][l, hi] + bufs[4][l, hi]))
                ob[l, lo] = ts
                ob[l, hi] = to
            return carry

        lax.fori_loop(0, L, ex_body, 0)
        pltpu.sync_copy(ob, out_h.at[pl.ds(base + cbase, K)])

    _pipeline(fire, drain, compute)


def _make_kT(mesh, params):
    scratch = (
        [pltpu.VMEM((BPW,), jnp.int32)] * 2
        + [pltpu.VMEM((BPW,), jnp.float32)] * 2
        + [pltpu.VMEM((K, DPAIR), jnp.float32)] * 14
        + [pltpu.SemaphoreType.DMA] * 2
    )
    return pl.kernel(
        _kT_body,
        out_type=jax.ShapeDtypeStruct((B, DPAIR), jnp.float32),
        mesh=mesh, scratch_types=scratch, compiler_params=params)


def _kF_body(s_h, o_h, r_h, ep_h, rf_h, ri_h, ts_h, to_h, out_h, *sc):
    si_v, oi_v, ri_v, out_v = sc[0:4]
    sets = [sc[4 + t * 6:4 + (t + 1) * 6] for t in range(2)]
    sems = sc[16:18]

    base = _worker_prelude(s_h, o_h, si_v, oi_v)
    pltpu.sync_copy(r_h.at[pl.ds(base, BPW)], ri_v)

    lane = lax.iota(jnp.int32, L)

    def _descs(cbase, t):
        bufs, sem = sets[t], sems[t]
        si = si_v.at[pl.ds(cbase, K)]
        oi = oi_v.at[pl.ds(cbase, K)]
        ri = ri_v.at[pl.ds(cbase, K)]
        rows = pl.ds(base + cbase, K)
        return [(ep_h.at[si], bufs[0], sem), (ep_h.at[oi], bufs[1], sem),
                (rf_h.at[ri], bufs[2], sem), (ri_h.at[ri], bufs[3], sem),
                (ts_h.at[rows], bufs[4], sem), (to_h.at[rows], bufs[5], sem)]

    def fire(cbase, t):
        for src, dst, sem in _descs(cbase, t):
            pltpu.async_copy(src, dst, sem)

    def drain(cbase, t):
        for src, dst, sem in _descs(cbase, t):
            pltpu.make_async_copy(src, dst, sem).wait()

    def compute(cbase, t):
        epS, epO, bRf, bRi, bTs, bTo = sets[t]

        def ex_body(l, svec):
            acc = jnp.zeros((L,), jnp.float32)
            for j in range(S_ES // L):
                lo = pl.ds(j * L, L)
                hi = pl.ds(S_ES + j * L, L)
                acc = acc + epS[l, lo] * bRf[l, lo] * epO[l, hi]
                acc = acc + bTs[l, lo] * bRf[l, hi] * bTo[l, hi]
                acc = acc + epO[l, lo] * bRi[l, lo] * epS[l, hi]
                acc = acc + bTs[l, hi] * bRi[l, hi] * bTo[l, lo]
            score = 0.5 * jnp.sum(acc)
            return jnp.where(lane == l, score, svec)

        svec = lax.fori_loop(0, L, ex_body, jnp.zeros((L,), jnp.float32))
        out_v[pl.ds(cbase, L)] = svec

    _pipeline(fire, drain, compute)
    pltpu.sync_copy(out_v, out_h.at[pl.ds(base, BPW)])


def _make_kF(mesh, params):
    scratch = (
        [pltpu.VMEM((BPW,), jnp.int32)] * 3
        + [pltpu.VMEM((BPW,), jnp.float32)]
        + [pltpu.VMEM((K, DPAIR), jnp.float32)] * 12
        + [pltpu.SemaphoreType.DMA] * 2
    )
    return pl.kernel(
        _kF_body,
        out_type=jax.ShapeDtypeStruct((B,), jnp.float32),
        mesh=mesh, scratch_types=scratch, compiler_params=params)


@jax.jit
def kernel(s, o, r, t, E_s, E_o, R_f, R_i,
           d_frq_s, d_frq_o, h_frq_s, h_frq_o,
           d_phi_s, d_phi_o, h_phi_s, h_phi_o,
           d_amp_s, d_amp_o, h_amp_s, h_amp_o):
    d = t[:, 0].astype(jnp.float32)
    h = t[:, 1].astype(jnp.float32)

    tpS = _make_tpose(3)(d_frq_s.T, h_frq_s.T, d_phi_s.T, h_phi_s.T,
                         d_amp_s.T, h_amp_s.T)
    tpO = _make_tpose(3)(d_frq_o.T, h_frq_o.T, d_phi_o.T, h_phi_o.T,
                         d_amp_o.T, h_amp_o.T)
    (epair,) = _make_tpose(1)(E_s.T, E_o.T)

    mesh = plsc.VectorSubcoreMesh(
        core_axis_name="c", subcore_axis_name="s", num_cores=NC, num_subcores=NS)
    params = pltpu.CompilerParams(
        needs_layout_passes=False, use_tc_tiling_on_sc=True)

    ts_rows = _make_kT(mesh, params)(s, o, d, h, *tpS)
    to_rows = _make_kT(mesh, params)(s, o, d, h, *tpO)
    return _make_kF(mesh, params)(s, o, r, epair, R_f, R_i, ts_rows, to_rows)


# speedup vs baseline: 1.1878x; 1.1874x over previous
import functools

import jax
import jax.numpy as jnp
from jax import lax
from jax.experimental import pallas as pl
from jax.experimental.pallas import tpu as pltpu
from jax.experimental.pallas import tpu_sc as plsc

B = 16384
S_ES = 64
DPAIR = 128
NC = 2
NS = 16
L = 16
NW = NC * NS
BPW = B // NW
K = 16
NCHUNK = BPW // K

_S0 = 9.99996152e-01
_S1 = -1.66647032e-01
_S2 = 8.31724544e-03
_S3 = -1.93765902e-04
_S4 = 2.19812516e-06
_TWO_PI = 6.283185307179586
_INV_2PI = 0.15915494309189535
_RND = 12582912.0


def _sin(x):
    k = (x * _INV_2PI + _RND) - _RND
    xr = x - k * _TWO_PI
    s = xr * xr
    p = _S4
    p = p * s + _S3
    p = p * s + _S2
    p = p * s + _S1
    p = p * s + _S0
    return p * xr


_EB = 2048
_NEB = (100000 + _EB - 1) // _EB


def _make_tpose(npair):
    def body(*refs):
        ins = refs[:2 * npair]
        outs = refs[2 * npair:]
        for p in range(npair):
            outs[p][:, 0:S_ES] = ins[2 * p][...].T
            outs[p][:, S_ES:DPAIR] = ins[2 * p + 1][...].T

    in_specs = [pl.BlockSpec((S_ES, _EB), lambda i: (0, i))
                for _ in range(2 * npair)]
    out_specs = [pl.BlockSpec((_EB, DPAIR), lambda i: (i, 0))
                 for _ in range(npair)]
    out_shape = [jax.ShapeDtypeStruct((100000, DPAIR), jnp.float32)
                 for _ in range(npair)]
    return pl.pallas_call(
        body, grid=(_NEB,), in_specs=in_specs, out_specs=out_specs,
        out_shape=out_shape,
        compiler_params=pltpu.CompilerParams(dimension_semantics=("parallel",)))


def _worker_prelude(s_h, o_h, si_v, oi_v):
    wid = lax.axis_index("s") * NC + lax.axis_index("c")
    base = wid * BPW
    pltpu.sync_copy(s_h.at[pl.ds(base, BPW)], si_v)
    pltpu.sync_copy(o_h.at[pl.ds(base, BPW)], oi_v)
    return base


def _pipeline(fire, drain, compute):
    fire(0, 0)

    def loop_body(g, carry):
        c0 = 2 * g
        c1 = 2 * g + 1
        fire(c1 * K, 1)
        drain(c0 * K, 0)
        compute(c0 * K, 0)
        nxt0 = jnp.minimum(c0 + 2, NCHUNK - 1) * K
        fire(nxt0, 0)
        drain(c1 * K, 1)
        compute(c1 * K, 1)
        return carry

    lax.fori_loop(0, NCHUNK // 2, loop_body, 0)
    drain((NCHUNK - 1) * K, 0)


def _kT_body(s_h, o_h, d_h, h_h, frq_h, phi_h, amp_h, out_h, *sc):
    si_v, oi_v, d_v, h_v = sc[0:4]
    sets = [sc[4 + t * 7:4 + (t + 1) * 7] for t in range(2)]
    sems = sc[18:20]

    base = _worker_prelude(s_h, o_h, si_v, oi_v)
    pltpu.sync_copy(d_h.at[pl.ds(base, BPW)], d_v)
    pltpu.sync_copy(h_h.at[pl.ds(base, BPW)], h_v)

    def _descs(cbase, t):
        bufs, sem = sets[t], sems[t]
        si = si_v.at[pl.ds(cbase, K)]
        oi = oi_v.at[pl.ds(cbase, K)]
        return [(frq_h.at[si], bufs[0], sem), (phi_h.at[si], bufs[1], sem),
                (amp_h.at[si], bufs[2], sem), (frq_h.at[oi], bufs[3], sem),
                (phi_h.at[oi], bufs[4], sem), (amp_h.at[oi], bufs[5], sem)]

    def fire(cbase, t):
        for src, dst, sem in _descs(cbase, t):
            pltpu.async_copy(src, dst, sem)

    def drain(cbase, t):
        for src, dst, sem in _descs(cbase, t):
            pltpu.make_async_copy(src, dst, sem).wait()

    def compute(cbase, t):
        bufs = sets[t]
        ob = bufs[6]

        def ex_body(l, carry):
            idxv = lax.broadcast(cbase + l, (L,))
            db = plsc.load_gather(d_v, [idxv])
            hb = plsc.load_gather(h_v, [idxv])
            for j in range(S_ES // L):
                lo = pl.ds(j * L, L)
                hi = pl.ds(S_ES + j * L, L)
                ts = (bufs[2][l, lo] * _sin(db * bufs[0][l, lo] + bufs[1][l, lo])
                      + bufs[2][l, hi] * _sin(hb * bufs[0][l, hi] + bufs[1][l, hi]))
                to = (bufs[5][l, lo] * _sin(db * bufs[3][l, lo] + bufs[4][l, lo])
                      + bufs[5][l, hi] * _sin(hb * bufs[3][l, hi] + bufs[4][l, hi]))
                ob[l, lo] = ts
                ob[l, hi] = to
            return carry

        lax.fori_loop(0, L, ex_body, 0)
        pltpu.sync_copy(ob, out_h.at[pl.ds(base + cbase, K)])

    _pipeline(fire, drain, compute)


def _make_kT(mesh, params):
    scratch = (
        [pltpu.VMEM((BPW,), jnp.int32)] * 2
        + [pltpu.VMEM((BPW,), jnp.float32)] * 2
        + [pltpu.VMEM((K, DPAIR), jnp.float32)] * 14
        + [pltpu.SemaphoreType.DMA] * 2
    )
    return pl.kernel(
        _kT_body,
        out_type=jax.ShapeDtypeStruct((B, DPAIR), jnp.float32),
        mesh=mesh, scratch_types=scratch, compiler_params=params)


def _kF_body(s_h, o_h, r_h, ep_h, rf_h, ri_h, ts_h, to_h, out_h, *sc):
    si_v, oi_v, ri_v, out_v = sc[0:4]
    sets = [sc[4 + t * 6:4 + (t + 1) * 6] for t in range(2)]
    sems = sc[16:18]

    base = _worker_prelude(s_h, o_h, si_v, oi_v)
    pltpu.sync_copy(r_h.at[pl.ds(base, BPW)], ri_v)

    lane = lax.iota(jnp.int32, L)

    def _descs(cbase, t):
        bufs, sem = sets[t], sems[t]
        si = si_v.at[pl.ds(cbase, K)]
        oi = oi_v.at[pl.ds(cbase, K)]
        ri = ri_v.at[pl.ds(cbase, K)]
        rows = pl.ds(base + cbase, K)
        return [(ep_h.at[si], bufs[0], sem), (ep_h.at[oi], bufs[1], sem),
                (rf_h.at[ri], bufs[2], sem), (ri_h.at[ri], bufs[3], sem),
                (ts_h.at[rows], bufs[4], sem), (to_h.at[rows], bufs[5], sem)]

    def fire(cbase, t):
        for src, dst, sem in _descs(cbase, t):
            pltpu.async_copy(src, dst, sem)

    def drain(cbase, t):
        for src, dst, sem in _descs(cbase, t):
            pltpu.make_async_copy(src, dst, sem).wait()

    def compute(cbase, t):
        epS, epO, bRf, bRi, bTs, bTo = sets[t]

        def ex_body(l, svec):
            acc = jnp.zeros((L,), jnp.float32)
            for j in range(S_ES // L):
                lo = pl.ds(j * L, L)
                hi = pl.ds(S_ES + j * L, L)
                acc = acc + epS[l, lo] * bRf[l, lo] * epO[l, hi]
                acc = acc + bTs[l, lo] * bRf[l, hi] * bTo[l, hi]
                acc = acc + epO[l, lo] * bRi[l, lo] * epS[l, hi]
                acc = acc + bTs[l, hi] * bRi[l, hi] * bTo[l, lo]
            score = 0.5 * jnp.sum(acc)
            return jnp.where(lane == l, score, svec)

        svec = lax.fori_loop(0, L, ex_body, jnp.zeros((L,), jnp.float32))
        out_v[pl.ds(cbase, L)] = svec

    _pipeline(fire, drain, compute)
    pltpu.sync_copy(out_v, out_h.at[pl.ds(base, BPW)])


def _make_kF(mesh, params):
    scratch = (
        [pltpu.VMEM((BPW,), jnp.int32)] * 3
        + [pltpu.VMEM((BPW,), jnp.float32)]
        + [pltpu.VMEM((K, DPAIR), jnp.float32)] * 12
        + [pltpu.SemaphoreType.DMA] * 2
    )
    return pl.kernel(
        _kF_body,
        out_type=jax.ShapeDtypeStruct((B,), jnp.float32),
        mesh=mesh, scratch_types=scratch, compiler_params=params)


@jax.jit
def kernel(s, o, r, t, E_s, E_o, R_f, R_i,
           d_frq_s, d_frq_o, h_frq_s, h_frq_o,
           d_phi_s, d_phi_o, h_phi_s, h_phi_o,
           d_amp_s, d_amp_o, h_amp_s, h_amp_o):
    d = t[:, 0].astype(jnp.float32)
    h = t[:, 1].astype(jnp.float32)

    tpS = _make_tpose(3)(d_frq_s.T, h_frq_s.T, d_phi_s.T, h_phi_s.T,
                         d_amp_s.T, h_amp_s.T)
    tpO = _make_tpose(3)(d_frq_o.T, h_frq_o.T, d_phi_o.T, h_phi_o.T,
                         d_amp_o.T, h_amp_o.T)
    (epair,) = _make_tpose(1)(E_s.T, E_o.T)

    mesh = plsc.VectorSubcoreMesh(
        core_axis_name="c", subcore_axis_name="s", num_cores=NC, num_subcores=NS)
    params = pltpu.CompilerParams(
        needs_layout_passes=False, use_tc_tiling_on_sc=True)

    ts_rows = _make_kT(mesh, params)(s, o, d, h, *tpS)
    to_rows = _make_kT(mesh, params)(s, o, d, h, *tpO)
    return _make_kF(mesh, params)(s, o, r, epair, R_f, R_i, ts_rows, to_rows)


# speedup vs baseline: 1.2761x; 1.0744x over previous
import functools

import jax
import jax.numpy as jnp
from jax import lax
from jax.experimental import pallas as pl
from jax.experimental.pallas import tpu as pltpu
from jax.experimental.pallas import tpu_sc as plsc

B = 16384
S_ES = 64
DPAIR = 128
NC = 2
NS = 16
L = 16
NW = NC * NS
BPW = B // NW
K = 16
NCHUNK = BPW // K

_S0 = 9.99996152e-01
_S1 = -1.66647032e-01
_S2 = 8.31724544e-03
_S3 = -1.93765902e-04
_S4 = 2.19812516e-06
_TWO_PI = 6.283185307179586
_INV_2PI = 0.15915494309189535
_RND = 12582912.0


def _sin(x):
    k = (x * _INV_2PI + _RND) - _RND
    xr = x - k * _TWO_PI
    s = xr * xr
    p = _S4
    p = p * s + _S3
    p = p * s + _S2
    p = p * s + _S1
    p = p * s + _S0
    return p * xr


_EB = 4096
_NEB = (100000 + _EB - 1) // _EB


def _make_tpose(npair):
    def body(*refs):
        ins = refs[:2 * npair]
        outs = refs[2 * npair:]
        for p in range(npair):
            outs[p][:, 0:S_ES] = ins[2 * p][...].T
            outs[p][:, S_ES:DPAIR] = ins[2 * p + 1][...].T

    in_specs = [pl.BlockSpec((S_ES, _EB), lambda i: (0, i))
                for _ in range(2 * npair)]
    out_specs = [pl.BlockSpec((_EB, DPAIR), lambda i: (i, 0))
                 for _ in range(npair)]
    out_shape = [jax.ShapeDtypeStruct((100000, DPAIR), jnp.float32)
                 for _ in range(npair)]
    return pl.pallas_call(
        body, grid=(_NEB,), in_specs=in_specs, out_specs=out_specs,
        out_shape=out_shape,
        compiler_params=pltpu.CompilerParams(dimension_semantics=("parallel",)))


def _worker_prelude(s_h, o_h, si_v, oi_v):
    wid = lax.axis_index("s") * NC + lax.axis_index("c")
    base = wid * BPW
    pltpu.sync_copy(s_h.at[pl.ds(base, BPW)], si_v)
    pltpu.sync_copy(o_h.at[pl.ds(base, BPW)], oi_v)
    return base


def _pipeline(fire, drain, compute):
    fire(0, 0)

    def loop_body(g, carry):
        c0 = 2 * g
        c1 = 2 * g + 1
        fire(c1 * K, 1)
        drain(c0 * K, 0)
        compute(c0 * K, 0)
        nxt0 = jnp.minimum(c0 + 2, NCHUNK - 1) * K
        fire(nxt0, 0)
        drain(c1 * K, 1)
        compute(c1 * K, 1)
        return carry

    lax.fori_loop(0, NCHUNK // 2, loop_body, 0)
    drain((NCHUNK - 1) * K, 0)


def _kT_body(s_h, o_h, d_h, h_h, frq_h, phi_h, amp_h, out_h, *sc):
    si_v, oi_v, d_v, h_v = sc[0:4]
    sets = [sc[4 + t * 7:4 + (t + 1) * 7] for t in range(2)]
    sems = sc[18:20]

    base = _worker_prelude(s_h, o_h, si_v, oi_v)
    pltpu.sync_copy(d_h.at[pl.ds(base, BPW)], d_v)
    pltpu.sync_copy(h_h.at[pl.ds(base, BPW)], h_v)

    def _descs(cbase, t):
        bufs, sem = sets[t], sems[t]
        si = si_v.at[pl.ds(cbase, K)]
        oi = oi_v.at[pl.ds(cbase, K)]
        return [(frq_h.at[si], bufs[0], sem), (phi_h.at[si], bufs[1], sem),
                (amp_h.at[si], bufs[2], sem), (frq_h.at[oi], bufs[3], sem),
                (phi_h.at[oi], bufs[4], sem), (amp_h.at[oi], bufs[5], sem)]

    def fire(cbase, t):
        for src, dst, sem in _descs(cbase, t):
            pltpu.async_copy(src, dst, sem)

    def drain(cbase, t):
        for src, dst, sem in _descs(cbase, t):
            pltpu.make_async_copy(src, dst, sem).wait()

    def compute(cbase, t):
        bufs = sets[t]
        ob = bufs[6]

        def ex_body(l, carry):
            idxv = lax.broadcast(cbase + l, (L,))
            db = plsc.load_gather(d_v, [idxv])
            hb = plsc.load_gather(h_v, [idxv])
            for j in range(S_ES // L):
                lo = pl.ds(j * L, L)
                hi = pl.ds(S_ES + j * L, L)
                ts = (bufs[2][l, lo] * _sin(db * bufs[0][l, lo] + bufs[1][l, lo])
                      + bufs[2][l, hi] * _sin(hb * bufs[0][l, hi] + bufs[1][l, hi]))
                to = (bufs[5][l, lo] * _sin(db * bufs[3][l, lo] + bufs[4][l, lo])
                      + bufs[5][l, hi] * _sin(hb * bufs[3][l, hi] + bufs[4][l, hi]))
                ob[l, lo] = ts
                ob[l, hi] = to
            return carry

        lax.fori_loop(0, L, ex_body, 0)
        pltpu.sync_copy(ob, out_h.at[pl.ds(base + cbase, K)])

    _pipeline(fire, drain, compute)


def _make_kT(mesh, params):
    scratch = (
        [pltpu.VMEM((BPW,), jnp.int32)] * 2
        + [pltpu.VMEM((BPW,), jnp.float32)] * 2
        + [pltpu.VMEM((K, DPAIR), jnp.float32)] * 14
        + [pltpu.SemaphoreType.DMA] * 2
    )
    return pl.kernel(
        _kT_body,
        out_type=jax.ShapeDtypeStruct((B, DPAIR), jnp.float32),
        mesh=mesh, scratch_types=scratch, compiler_params=params)


def _kF_body(s_h, o_h, r_h, ep_h, rf_h, ri_h, ts_h, to_h, out_h, *sc):
    si_v, oi_v, ri_v, out_v = sc[0:4]
    sets = [sc[4 + t * 6:4 + (t + 1) * 6] for t in range(2)]
    sems = sc[16:18]

    base = _worker_prelude(s_h, o_h, si_v, oi_v)
    pltpu.sync_copy(r_h.at[pl.ds(base, BPW)], ri_v)

    lane = lax.iota(jnp.int32, L)

    def _descs(cbase, t):
        bufs, sem = sets[t], sems[t]
        si = si_v.at[pl.ds(cbase, K)]
        oi = oi_v.at[pl.ds(cbase, K)]
        ri = ri_v.at[pl.ds(cbase, K)]
        rows = pl.ds(base + cbase, K)
        return [(ep_h.at[si], bufs[0], sem), (ep_h.at[oi], bufs[1], sem),
                (rf_h.at[ri], bufs[2], sem), (ri_h.at[ri], bufs[3], sem),
                (ts_h.at[rows], bufs[4], sem), (to_h.at[rows], bufs[5], sem)]

    def fire(cbase, t):
        for src, dst, sem in _descs(cbase, t):
            pltpu.async_copy(src, dst, sem)

    def drain(cbase, t):
        for src, dst, sem in _descs(cbase, t):
            pltpu.make_async_copy(src, dst, sem).wait()

    def compute(cbase, t):
        epS, epO, bRf, bRi, bTs, bTo = sets[t]

        def ex_body(l, svec):
            acc = jnp.zeros((L,), jnp.float32)
            for j in range(S_ES // L):
                lo = pl.ds(j * L, L)
                hi = pl.ds(S_ES + j * L, L)
                acc = acc + epS[l, lo] * bRf[l, lo] * epO[l, hi]
                acc = acc + bTs[l, lo] * bRf[l, hi] * bTo[l, hi]
                acc = acc + epO[l, lo] * bRi[l, lo] * epS[l, hi]
                acc = acc + bTs[l, hi] * bRi[l, hi] * bTo[l, lo]
            score = 0.5 * jnp.sum(acc)
            return jnp.where(lane == l, score, svec)

        svec = lax.fori_loop(0, L, ex_body, jnp.zeros((L,), jnp.float32))
        out_v[pl.ds(cbase, L)] = svec

    _pipeline(fire, drain, compute)
    pltpu.sync_copy(out_v, out_h.at[pl.ds(base, BPW)])


def _make_kF(mesh, params):
    scratch = (
        [pltpu.VMEM((BPW,), jnp.int32)] * 3
        + [pltpu.VMEM((BPW,), jnp.float32)]
        + [pltpu.VMEM((K, DPAIR), jnp.float32)] * 12
        + [pltpu.SemaphoreType.DMA] * 2
    )
    return pl.kernel(
        _kF_body,
        out_type=jax.ShapeDtypeStruct((B,), jnp.float32),
        mesh=mesh, scratch_types=scratch, compiler_params=params)


@jax.jit
def kernel(s, o, r, t, E_s, E_o, R_f, R_i,
           d_frq_s, d_frq_o, h_frq_s, h_frq_o,
           d_phi_s, d_phi_o, h_phi_s, h_phi_o,
           d_amp_s, d_amp_o, h_amp_s, h_amp_o):
    d = t[:, 0].astype(jnp.float32)
    h = t[:, 1].astype(jnp.float32)

    tpS = _make_tpose(3)(d_frq_s.T, h_frq_s.T, d_phi_s.T, h_phi_s.T,
                         d_amp_s.T, h_amp_s.T)
    tpO = _make_tpose(3)(d_frq_o.T, h_frq_o.T, d_phi_o.T, h_phi_o.T,
                         d_amp_o.T, h_amp_o.T)
    (epair,) = _make_tpose(1)(E_s.T, E_o.T)

    mesh = plsc.VectorSubcoreMesh(
        core_axis_name="c", subcore_axis_name="s", num_cores=NC, num_subcores=NS)
    params = pltpu.CompilerParams(
        needs_layout_passes=False, use_tc_tiling_on_sc=True)

    ts_rows = _make_kT(mesh, params)(s, o, d, h, *tpS)
    to_rows = _make_kT(mesh, params)(s, o, d, h, *tpO)
    return _make_kF(mesh, params)(s, o, r, epair, R_f, R_i, ts_rows, to_rows)
